# baseline (speedup 1.0000x reference)
import jax
import jax.numpy as jnp
from jax.experimental import pallas as pl

_NPOINTS = [1024, 256, 64, 16]
_RADII = [0.1, 0.2, 0.4, 0.8]
_NSAMPLE = 32


def _mlp_matmul_kernel(x_ref, w_ref, o_ref):
    o_ref[0] = jnp.dot(x_ref[0], w_ref[...], preferred_element_type=jnp.float32)


def _linear_pallas(x, W):
    B, M, C = x.shape
    O = W.shape[0]
    Cp = ((C + 127) // 128) * 128
    if Cp != C:
        x = jnp.pad(x, ((0, 0), (0, 0), (0, Cp - C)))
        W = jnp.pad(W, ((0, 0), (0, Cp - C)))
    Wt = jnp.transpose(W)
    TM = min(M, 4096)
    return pl.pallas_call(
        _mlp_matmul_kernel,
        grid=(B, M // TM),
        in_specs=[
            pl.BlockSpec((1, TM, Cp), lambda i, j: (i, j, 0)),
            pl.BlockSpec((Cp, O), lambda i, j: (0, 0)),
        ],
        out_specs=pl.BlockSpec((1, TM, O), lambda i, j: (i, j, 0)),
        out_shape=jax.ShapeDtypeStruct((B, M, O), jnp.float32),
    )(x, Wt)


def _square_distance(src, dst):
    d = -2.0 * jnp.einsum('bmc,bnc->bmn', src, dst)
    d = d + jnp.sum(src ** 2, axis=-1)[:, :, None]
    d = d + jnp.sum(dst ** 2, axis=-1)[:, None, :]
    return d


def _index_points(points, idx):
    b = points.shape[0]
    batch = jnp.arange(b).reshape((b,) + (1,) * (idx.ndim - 1))
    return points[batch, idx]


def _farthest_point_sample(xyz, npoint):
    b, n, _ = xyz.shape

    def body(i, state):
        centroids, distance, farthest = state
        centroids = centroids.at[:, i].set(farthest)
        centroid = xyz[jnp.arange(b), farthest][:, None, :]
        dist = jnp.sum((xyz - centroid) ** 2, axis=-1)
        distance = jnp.minimum(distance, dist)
        farthest = jnp.argmax(distance, axis=-1).astype(jnp.int32)
        return centroids, distance, farthest

    centroids = jnp.zeros((b, npoint), dtype=jnp.int32)
    distance = jnp.full((b, n), 1e10, dtype=xyz.dtype)
    farthest = jnp.zeros((b,), dtype=jnp.int32)
    centroids, _, _ = jax.lax.fori_loop(0, npoint, body, (centroids, distance, farthest))
    return centroids


def _query_ball_point(radius, nsample, xyz, new_xyz):
    b, n, _ = xyz.shape
    s = new_xyz.shape[1]
    sqrdists = _square_distance(new_xyz, xyz)
    group_idx = jnp.broadcast_to(jnp.arange(n, dtype=jnp.int32), (b, s, n))
    group_idx = jnp.where(sqrdists > radius ** 2, n, group_idx)
    group_idx = jnp.sort(group_idx, axis=-1)[:, :, :nsample]
    group_first = group_idx[:, :, :1]
    group_idx = jnp.where(group_idx == n, group_first, group_idx)
    return group_idx


def _sa_layer(xyz, points, npoint, radius, nsample, layer_params):
    xyz_t = jnp.transpose(xyz, (0, 2, 1))
    points_t = jnp.transpose(points, (0, 2, 1))
    xyz_sg = jax.lax.stop_gradient(xyz_t)
    fps_idx = _farthest_point_sample(xyz_sg, npoint)
    new_xyz = _index_points(xyz_t, fps_idx)
    idx = _query_ball_point(radius, nsample, xyz_sg, jax.lax.stop_gradient(new_xyz))
    grouped_xyz = _index_points(xyz_t, idx)
    grouped_xyz_norm = grouped_xyz - new_xyz[:, :, None, :]
    grouped_points = _index_points(points_t, idx)
    h = jnp.concatenate([grouped_xyz_norm, grouped_points], axis=-1)
    b, s, k, c = h.shape
    h2 = h.reshape(b, s * k, c)
    for (W, bb, gamma, beta) in layer_params:
        y = _linear_pallas(h2, W) + bb
        mean = jnp.mean(y, axis=(0, 1), keepdims=True)
        var = jnp.var(y, axis=(0, 1), keepdims=True)
        y = (y - mean) / jnp.sqrt(var + 1e-5) * gamma + beta
        h2 = jax.nn.relu(y)
    h = h2.reshape(b, s, k, -1)
    new_points = jnp.max(h, axis=2)
    return jnp.transpose(new_xyz, (0, 2, 1)), jnp.transpose(new_points, (0, 2, 1))


def kernel(xyz, params):
    l0_xyz = xyz[:, :3, :]
    lx, lp = l0_xyz, xyz
    xs = [l0_xyz]
    ps = []
    for li in range(4):
        lx, lp = _sa_layer(lx, lp, _NPOINTS[li], _RADII[li], _NSAMPLE, params[li])
        xs.append(lx)
        ps.append(lp)
    return tuple(xs) + tuple(ps)


# speedup vs baseline: 1.0054x; 1.0054x over previous
import jax
import jax.numpy as jnp
from jax.experimental import pallas as pl

_NPOINTS = [1024, 256, 64, 16]
_RADII = [0.1, 0.2, 0.4, 0.8]
_NSAMPLE = 32


def _mlp_matmul_kernel(x_ref, w_ref, o_ref):
    o_ref[0] = jnp.dot(x_ref[0], w_ref[...], preferred_element_type=jnp.float32)


def _linear_pallas(x, W):
    B, M, C = x.shape
    O = W.shape[0]
    Wt = jnp.transpose(W)
    TM = min(M, 4096)
    return pl.pallas_call(
        _mlp_matmul_kernel,
        grid=(B, M // TM),
        in_specs=[
            pl.BlockSpec((1, TM, C), lambda i, j: (i, j, 0)),
            pl.BlockSpec((C, O), lambda i, j: (0, 0)),
        ],
        out_specs=pl.BlockSpec((1, TM, O), lambda i, j: (i, j, 0)),
        out_shape=jax.ShapeDtypeStruct((B, M, O), jnp.float32),
    )(x, Wt)


def _square_distance(src, dst):
    d = -2.0 * jnp.einsum('bmc,bnc->bmn', src, dst)
    d = d + jnp.sum(src ** 2, axis=-1)[:, :, None]
    d = d + jnp.sum(dst ** 2, axis=-1)[:, None, :]
    return d


def _index_points(points, idx):
    b = points.shape[0]
    batch = jnp.arange(b).reshape((b,) + (1,) * (idx.ndim - 1))
    return points[batch, idx]


def _farthest_point_sample(xyz, npoint):
    b, n, _ = xyz.shape

    def body(i, state):
        centroids, distance, farthest = state
        centroids = centroids.at[:, i].set(farthest)
        centroid = xyz[jnp.arange(b), farthest][:, None, :]
        dist = jnp.sum((xyz - centroid) ** 2, axis=-1)
        distance = jnp.minimum(distance, dist)
        farthest = jnp.argmax(distance, axis=-1).astype(jnp.int32)
        return centroids, distance, farthest

    centroids = jnp.zeros((b, npoint), dtype=jnp.int32)
    distance = jnp.full((b, n), 1e10, dtype=xyz.dtype)
    farthest = jnp.zeros((b,), dtype=jnp.int32)
    centroids, _, _ = jax.lax.fori_loop(0, npoint, body, (centroids, distance, farthest))
    return centroids


def _query_ball_point(radius, nsample, xyz, new_xyz):
    b, n, _ = xyz.shape
    s = new_xyz.shape[1]
    sqrdists = _square_distance(new_xyz, xyz)
    group_idx = jnp.broadcast_to(jnp.arange(n, dtype=jnp.int32), (b, s, n))
    group_idx = jnp.where(sqrdists > radius ** 2, n, group_idx)
    group_idx = jnp.sort(group_idx, axis=-1)[:, :, :nsample]
    group_first = group_idx[:, :, :1]
    group_idx = jnp.where(group_idx == n, group_first, group_idx)
    return group_idx


def _sa_layer(xyz, points, npoint, radius, nsample, layer_params):
    xyz_t = jnp.transpose(xyz, (0, 2, 1))
    points_t = jnp.transpose(points, (0, 2, 1))
    xyz_sg = jax.lax.stop_gradient(xyz_t)
    fps_idx = _farthest_point_sample(xyz_sg, npoint)
    new_xyz = _index_points(xyz_t, fps_idx)
    idx = _query_ball_point(radius, nsample, xyz_sg, jax.lax.stop_gradient(new_xyz))
    grouped_xyz = _index_points(xyz_t, idx)
    grouped_xyz_norm = grouped_xyz - new_xyz[:, :, None, :]
    grouped_points = _index_points(points_t, idx)
    h = jnp.concatenate([grouped_xyz_norm, grouped_points], axis=-1)
    b, s, k, c = h.shape
    h2 = h.reshape(b, s * k, c)
    for (W, bb, gamma, beta) in layer_params:
        y = _linear_pallas(h2, W) + bb
        mean = jnp.mean(y, axis=(0, 1), keepdims=True)
        var = jnp.var(y, axis=(0, 1), keepdims=True)
        y = (y - mean) / jnp.sqrt(var + 1e-5) * gamma + beta
        h2 = jax.nn.relu(y)
    h = h2.reshape(b, s, k, -1)
    new_points = jnp.max(h, axis=2)
    return jnp.transpose(new_xyz, (0, 2, 1)), jnp.transpose(new_points, (0, 2, 1))


def kernel(xyz, params):
    l0_xyz = xyz[:, :3, :]
    lx, lp = l0_xyz, xyz
    xs = [l0_xyz]
    ps = []
    for li in range(4):
        lx, lp = _sa_layer(lx, lp, _NPOINTS[li], _RADII[li], _NSAMPLE, params[li])
        xs.append(lx)
        ps.append(lp)
    return tuple(xs) + tuple(ps)
